# baseline (speedup 1.0000x reference)
import functools
import numpy as np
import jax
import jax.numpy as jnp
from jax import lax
from jax.experimental import pallas as pl
from jax.experimental.pallas import tpu as pltpu
from jax.experimental.pallas import tpu_sc as plsc

B_TOTAL = 16384
D = 128
NW = 32
R = B_TOTAL // NW
_TINY = np.float32(np.finfo(np.float32).tiny)
_LN2 = np.float32(0.6931471805599453)

_R0 = (13, 15, 26, 6)
_R1 = (17, 29, 16, 24)


def _threefry_bits(cnt):
    k0 = jnp.uint32(0)
    k1 = jnp.uint32(42)
    k2 = k0 ^ k1 ^ jnp.uint32(0x1BD11BDA)

    def four_rounds(x0, x1, rots):
        for r in rots:
            x0 = x0 + x1
            x1 = (x1 << jnp.uint32(r)) | (x1 >> jnp.uint32(32 - r))
            x1 = x0 ^ x1
        return x0, x1

    x0 = jnp.full_like(cnt, k0)
    x1 = cnt + k1
    x0, x1 = four_rounds(x0, x1, _R0)
    x0 = x0 + k1
    x1 = x1 + (k2 + jnp.uint32(1))
    x0, x1 = four_rounds(x0, x1, _R1)
    x0 = x0 + k2
    x1 = x1 + (k0 + jnp.uint32(2))
    x0, x1 = four_rounds(x0, x1, _R0)
    x0 = x0 + k0
    x1 = x1 + (k1 + jnp.uint32(3))
    x0, x1 = four_rounds(x0, x1, _R1)
    x0 = x0 + k1
    x1 = x1 + (k2 + jnp.uint32(4))
    x0, x1 = four_rounds(x0, x1, _R0)
    x0 = x0 + k2
    x1 = x1 + (k0 + jnp.uint32(5))
    return x0 ^ x1


def _sc_log(x):
    bits = lax.bitcast_convert_type(x, jnp.int32)
    e = (bits >> 23) - 127
    m = lax.bitcast_convert_type(
        (bits & jnp.int32(0x007FFFFF)) | jnp.int32(0x3F800000), jnp.float32)
    big = m > jnp.float32(1.4142135)
    m = jnp.where(big, m * jnp.float32(0.5), m)
    ef = (e + jnp.where(big, jnp.int32(1), jnp.int32(0))).astype(jnp.float32)
    t = m - jnp.float32(1.0)
    s = t / (jnp.float32(2.0) + t)
    z = s * s
    p = jnp.float32(1.0) + z * (jnp.float32(1.0 / 3.0) + z * (
        jnp.float32(0.2) + z * (jnp.float32(1.0 / 7.0) + z * jnp.float32(1.0 / 9.0))))
    return ef * _LN2 + jnp.float32(2.0) * s * p


def _threefry_bits3(cnts):
    k0 = jnp.uint32(0)
    k1 = jnp.uint32(42)
    k2 = k0 ^ k1 ^ jnp.uint32(0x1BD11BDA)

    def four_rounds(st, rots):
        for r in rots:
            st = [(x0 + x1, x1) for (x0, x1) in st]
            st = [(x0, (x1 << jnp.uint32(r)) | (x1 >> jnp.uint32(32 - r)))
                  for (x0, x1) in st]
            st = [(x0, x0 ^ x1) for (x0, x1) in st]
        return st

    def bump(st, ka, kb, i):
        return [(x0 + ka, x1 + (kb + jnp.uint32(i))) for (x0, x1) in st]

    st = [(jnp.full_like(c, k0), c + k1) for c in cnts]
    st = bump(four_rounds(st, _R0), k1, k2, 1)
    st = bump(four_rounds(st, _R1), k2, k0, 2)
    st = bump(four_rounds(st, _R0), k0, k1, 3)
    st = bump(four_rounds(st, _R1), k1, k2, 4)
    st = bump(four_rounds(st, _R0), k2, k0, 5)
    return [x0 ^ x1 for (x0, x1) in st]


def _gumbel3(cnts):
    outs = []
    for bits in _threefry_bits3(cnts):
        fb = (bits >> jnp.uint32(9)) | jnp.uint32(0x3F800000)
        f = lax.bitcast_convert_type(fb, jnp.float32) - jnp.float32(1.0)
        u = jnp.maximum(_TINY, f + _TINY)
        outs.append(-_sc_log(-_sc_log(u)))
    return outs


def _rne_bf16(v):
    bits = lax.bitcast_convert_type(v, jnp.int32)
    r = bits + jnp.int32(0x7FFF) + ((bits >> 16) & jnp.int32(1))
    return lax.bitcast_convert_type(r & jnp.int32(-65536), jnp.float32)


def _sc_body(x_hbm, wb_hbm, b_hbm,
             act_hbm, lpi_hbm, neg_hbm, hp_hbm,
             xv, wbv, bv, l0v, l1v, l2v, actv, lpiv, negv, hpv):
    wid = lax.axis_index("s") * 2 + lax.axis_index("c")
    base = wid * R
    pltpu.sync_copy(x_hbm.at[pl.ds(base * D, R * D)], xv)
    pltpu.sync_copy(wb_hbm, wbv)
    pltpu.sync_copy(b_hbm, bv)

    lane = lax.iota(jnp.int32, 16)

    bvec = bv[...]
    bs = [bvec[j] for j in range(3)]
    lrefs = (l0v, l1v, l2v)

    G = 8
    for t in range(4):
        gbase = [(t * G * 16 + g * 16 + lane) * D for g in range(G)]
        zero = jnp.zeros((16,), jnp.float32)
        init = tuple(zero for _ in range(3 * G))

        @plsc.parallel_loop(0, D, 1, unroll=2, carry=init)
        def dloop(d, accs, gbase=gbase):
            ws = [wbv[j, d] for j in range(3)]
            accs = list(accs)
            for g in range(G):
                xg = _rne_bf16(plsc.load_gather(xv, [gbase[g] + d]))
                for j in range(3):
                    accs[g * 3 + j] = accs[g * 3 + j] + xg * ws[j]
            return tuple(accs)

        accs = dloop
        for g in range(G):
            off = t * G * 16 + g * 16
            for j in range(3):
                lrefs[j][pl.ds(off, 16)] = accs[g * 3 + j] + bs[j]

    @plsc.parallel_loop(0, R // 16, 1, unroll=2)
    def samp_group(gi):
        off = gi * 16
        l0 = l0v[pl.ds(off, 16)]
        l1 = l1v[pl.ds(off, 16)]
        l2 = l2v[pl.ds(off, 16)]
        row3 = ((base + off) + lane) * 3
        g0, g1, g2 = _gumbel3([row3.astype(jnp.uint32),
                               (row3 + 1).astype(jnp.uint32),
                               (row3 + 2).astype(jnp.uint32)])
        y0 = g0 + l0
        y1 = g1 + l1
        y2 = g2 + l2
        a = jnp.where(y1 > y0, jnp.int32(1), jnp.int32(0))
        a = jnp.where(y2 > jnp.maximum(y0, y1), jnp.int32(2), a)
        m = jnp.maximum(jnp.maximum(l0, l1), l2)
        e0 = jnp.exp(l0 - m)
        e1 = jnp.exp(l1 - m)
        e2 = jnp.exp(l2 - m)
        ls = _sc_log(e0 + e1 + e2)
        lp0 = (l0 - m) - ls
        lp1 = (l1 - m) - ls
        lp2 = (l2 - m) - ls
        lpi = jnp.where(a == 0, lp0, jnp.where(a == 1, lp1, lp2))
        hp = jnp.exp(lp1)
        neg = -_sc_log(hp)
        actv[pl.ds(off, 16)] = a
        lpiv[pl.ds(off, 16)] = lpi
        negv[pl.ds(off, 16)] = neg
        hpv[pl.ds(off, 16)] = hp

    pltpu.sync_copy(actv, act_hbm.at[pl.ds(base, R)])
    pltpu.sync_copy(lpiv, lpi_hbm.at[pl.ds(base, R)])
    pltpu.sync_copy(negv, neg_hbm.at[pl.ds(base, R)])
    pltpu.sync_copy(hpv, hp_hbm.at[pl.ds(base, R)])


def kernel(x, W, b):
    wrne = W.astype(jnp.bfloat16).astype(jnp.float32)
    wb = jnp.broadcast_to(wrne.T[:, :, None], (3, D, 16))
    x1 = x.reshape(-1)
    b16 = jnp.pad(b, (0, 13))
    mesh = plsc.VectorSubcoreMesh(core_axis_name="c", subcore_axis_name="s")
    sc = functools.partial(
        pl.kernel,
        out_type=[
            jax.ShapeDtypeStruct((B_TOTAL,), jnp.int32),
            jax.ShapeDtypeStruct((B_TOTAL,), jnp.float32),
            jax.ShapeDtypeStruct((B_TOTAL,), jnp.float32),
            jax.ShapeDtypeStruct((B_TOTAL,), jnp.float32),
        ],
        mesh=mesh,
        compiler_params=pltpu.CompilerParams(needs_layout_passes=False),
        scratch_types=[
            pltpu.VMEM((R * D,), jnp.float32),
            pltpu.VMEM((3, D, 16), jnp.float32),
            pltpu.VMEM((16,), jnp.float32),
            pltpu.VMEM((R,), jnp.float32),
            pltpu.VMEM((R,), jnp.float32),
            pltpu.VMEM((R,), jnp.float32),
            pltpu.VMEM((R,), jnp.int32),
            pltpu.VMEM((R,), jnp.float32),
            pltpu.VMEM((R,), jnp.float32),
            pltpu.VMEM((R,), jnp.float32),
        ],
    )(_sc_body)
    act, lpi, neg, hp = sc(x1, wb, b16)
    rs = lambda t: t.reshape(B_TOTAL, 1)
    return (rs(act), rs(lpi), rs(neg), rs(hp))


# speedup vs baseline: 11.5267x; 11.5267x over previous
import numpy as np
import jax
import jax.numpy as jnp
from jax.experimental import pallas as pl

B_TOTAL = 16384
D = 128
BLK = 8192
SUB = BLK // 128
_TINY = np.float32(np.finfo(np.float32).tiny)

_R0 = (13, 15, 26, 6)
_R1 = (17, 29, 16, 24)


def _threefry_bits(cnt):
    k0 = jnp.uint32(0)
    k1 = jnp.uint32(42)
    k2 = k0 ^ k1 ^ jnp.uint32(0x1BD11BDA)

    def four_rounds(x0, x1, rots):
        for r in rots:
            x0 = x0 + x1
            x1 = (x1 << jnp.uint32(r)) | (x1 >> jnp.uint32(32 - r))
            x1 = x0 ^ x1
        return x0, x1

    x0 = jnp.full_like(cnt, k0)
    x1 = cnt + k1
    x0, x1 = four_rounds(x0, x1, _R0)
    x0 = x0 + k1
    x1 = x1 + (k2 + jnp.uint32(1))
    x0, x1 = four_rounds(x0, x1, _R1)
    x0 = x0 + k2
    x1 = x1 + (k0 + jnp.uint32(2))
    x0, x1 = four_rounds(x0, x1, _R0)
    x0 = x0 + k0
    x1 = x1 + (k1 + jnp.uint32(3))
    x0, x1 = four_rounds(x0, x1, _R1)
    x0 = x0 + k1
    x1 = x1 + (k2 + jnp.uint32(4))
    x0, x1 = four_rounds(x0, x1, _R0)
    x0 = x0 + k2
    x1 = x1 + (k0 + jnp.uint32(5))
    return x0 ^ x1


def _gumbel(bits):
    fb = (bits >> jnp.uint32(9)) | jnp.uint32(0x3F800000)
    f = jax.lax.bitcast_convert_type(fb, jnp.float32) - jnp.float32(1.0)
    u = jnp.maximum(_TINY, f + _TINY)
    return -jnp.log(-jnp.log(u))


def _body(x_ref, w_ref, b_ref, act_ref, lpi_ref, neg_ref, hp_ref):
    blk = pl.program_id(0)
    x = x_ref[...]
    w = w_ref[...]
    lt = jax.lax.dot_general(w, x, (((0,), (1,)), ((), ())),
                             preferred_element_type=jnp.float32)

    s_iota = jax.lax.broadcasted_iota(jnp.int32, (SUB, 128), 0)
    l_iota = jax.lax.broadcasted_iota(jnp.int32, (SUB, 128), 1)
    r3 = (s_iota * 128 + l_iota) * 3 + blk * (BLK * 3)

    lg = []
    ys = []
    for j in range(3):
        lj = jnp.reshape(lt[j:j + 1, :], (SUB, 128)) + b_ref[0, j]
        g = _gumbel(_threefry_bits((r3 + j).astype(jnp.uint32)))
        lg.append(lj)
        ys.append(g + lj)

    l0, l1, l2 = lg
    m = jnp.maximum(jnp.maximum(l0, l1), l2)
    e0 = jnp.exp(l0 - m)
    e1 = jnp.exp(l1 - m)
    e2 = jnp.exp(l2 - m)
    ls = jnp.log(e0 + e1 + e2)
    lp0 = (l0 - m) - ls
    lp1 = (l1 - m) - ls
    lp2 = (l2 - m) - ls

    y0, y1, y2 = ys
    a = jnp.where(y1 > y0, jnp.int32(1), jnp.int32(0))
    a = jnp.where(y2 > jnp.maximum(y0, y1), jnp.int32(2), a)

    lpi = jnp.where(a == 0, lp0, jnp.where(a == 1, lp1, lp2))
    hp = jnp.exp(lp1)
    neg = -jnp.log(hp)

    act_ref[...] = a
    lpi_ref[...] = lpi
    neg_ref[...] = neg
    hp_ref[...] = hp


def kernel(x, W, b):
    bp = b.reshape(1, 3)
    grid = (B_TOTAL // BLK,)
    out_rows = B_TOTAL // 128
    act, lpi, neg, hp = pl.pallas_call(
        _body,
        grid=grid,
        in_specs=[
            pl.BlockSpec((BLK, D), lambda i: (i, 0)),
            pl.BlockSpec((D, 3), lambda i: (0, 0)),
            pl.BlockSpec((1, 3), lambda i: (0, 0)),
        ],
        out_specs=[pl.BlockSpec((SUB, 128), lambda i: (i, 0))] * 4,
        out_shape=[
            jax.ShapeDtypeStruct((out_rows, 128), jnp.int32),
            jax.ShapeDtypeStruct((out_rows, 128), jnp.float32),
            jax.ShapeDtypeStruct((out_rows, 128), jnp.float32),
            jax.ShapeDtypeStruct((out_rows, 128), jnp.float32),
        ],
    )(x, W, bp)
    rs = lambda t: t.reshape(B_TOTAL, 1)
    return (rs(act), rs(lpi), rs(neg), rs(hp))
